# baseline (speedup 1.0000x reference)
import functools

import jax
import jax.numpy as jnp
from jax import lax
from jax.experimental import pallas as pl
from jax.experimental.pallas import tpu as pltpu
from jax.experimental.pallas import tpu_sc as plsc

NG = 4096
NNZ = 8192
NWK = 32
NCAND = 10 * NNZ
CAND_W = NCAND // NWK
SEG = CAND_W + 32


P_CH = 128


def _apply_pairs(x_ref, w_ref, cnt_ref, pk_ref, y_ref, xg_ref, cm_ref, ci, co):
    def seg(w, carry):
        cw = cnt_ref[w]
        n_ch = (cw + P_CH - 1) // P_CH

        def chunk(ch, carry2):
            m = jnp.minimum(cw - ch * P_CH, P_CH)

            def gather(p, carry3):
                code = pk_ref[w, ch * P_CH + p]
                j = (code >> 13) & (NNZ - 1)
                xg_ref[pl.ds(p, 1), :] = x_ref[pl.ds(j, 1), :]
                return carry3

            lax.fori_loop(0, m, gather, 0)
            for k in range(9):
                cm_ref[pl.ds(k * P_CH, P_CH), :] = jnp.dot(
                    xg_ref[...], w_ref[k], preferred_element_type=jnp.float32)

            def scatter(p, carry3):
                code = pk_ref[w, ch * P_CH + p]
                i = code & (NNZ - 1)
                kk = code >> 26
                neg = kk >= 9
                k = jnp.where(neg, kk - 9, kk)
                s = jnp.where(neg, jnp.float32(-1.0), jnp.float32(1.0))
                y_ref[pl.ds(i, 1), :] = (y_ref[pl.ds(i, 1), :]
                                         + s * cm_ref[pl.ds(k * P_CH + p, 1), :])
                return carry3

            lax.fori_loop(0, m, scatter, 0)
            return carry2

        lax.fori_loop(0, n_ch, chunk, 0)
        return carry

    lax.fori_loop(0, NWK, seg, 0)


def _prelu_ref(y_ref, a_ref):
    a = a_ref[0]
    yv = y_ref[...]
    y_ref[...] = jnp.where(yv >= 0, yv, a * yv)


def _l12_body(av_ref, w1_ref, b1_ref, a1_ref, w2_ref, b2_ref, a2_ref,
              cnt_ref, pk_ref, y_ref, x_s, xg_s, cm_s):
    x1 = av_ref[...] * w1_ref[...] + b1_ref[...]
    a1 = a1_ref[0]
    x_s[...] = jnp.where(x1 >= 0, x1, a1 * x1)
    y_ref[...] = (jnp.dot(x_s[...], w2_ref[4], preferred_element_type=jnp.float32)
                  + b2_ref[...])
    _apply_pairs(x_s, w2_ref, cnt_ref, pk_ref, y_ref, xg_s, cm_s, 64, 256)
    _prelu_ref(y_ref, a2_ref)


def _conv_body(x_ref, w_ref, b_ref, a_ref, cnt_ref, pk_ref, y_ref, xg_s, cm_s,
               *, ci, co):
    y_ref[...] = (jnp.dot(x_ref[...], w_ref[4], preferred_element_type=jnp.float32)
                  + b_ref[...])
    _apply_pairs(x_ref, w_ref, cnt_ref, pk_ref, y_ref, xg_s, cm_s, ci, co)
    _prelu_ref(y_ref, a_ref)


_SMEM = pl.BlockSpec(memory_space=pltpu.SMEM)


def _vspec():
    return pl.BlockSpec(memory_space=pltpu.VMEM)


def _layer12(av, w1, b1, a1, w2, b2, a2, cnt, pk):
    return pl.pallas_call(
        _l12_body,
        out_shape=jax.ShapeDtypeStruct((NNZ, 256), jnp.float32),
        in_specs=[_vspec(), _vspec(), _vspec(), _SMEM,
                  _vspec(), _vspec(), _SMEM, _SMEM, _SMEM],
        out_specs=_vspec(),
        scratch_shapes=[pltpu.VMEM((NNZ, 64), jnp.float32),
                        pltpu.VMEM((P_CH, 64), jnp.float32),
                        pltpu.VMEM((9 * P_CH, 256), jnp.float32)],
    )(av, w1, b1, a1, w2, b2, a2, cnt, pk)


def _conv_layer(x, w, b, a, cnt, pk, ci, co):
    return pl.pallas_call(
        functools.partial(_conv_body, ci=ci, co=co),
        out_shape=jax.ShapeDtypeStruct((NNZ, co), jnp.float32),
        in_specs=[_vspec(), _vspec(), _vspec(), _SMEM, _SMEM, _SMEM],
        out_specs=_vspec(),
        scratch_shapes=[pltpu.VMEM((P_CH, ci), jnp.float32),
                        pltpu.VMEM((9 * P_CH, co), jnp.float32)],
    )(x, w, b, a, cnt, pk)


WORDS = NG * NG
W_WORDS = WORDS // NWK
PASS_WORDS = 32768
NPASS = W_WORDS // PASS_WORDS
NENT = 2 * NNZ


def _scatter_body(addr_hbm, val_hbm, zeros_hbm, out_hbm,
                  addr_v, val_v, ca_v, cv_v, buf):
    cid = lax.axis_index("c")
    sid = lax.axis_index("s")
    wid = sid * 2 + cid
    pltpu.sync_copy(addr_hbm, addr_v)
    pltpu.sync_copy(val_hbm, val_v)
    base0 = wid * W_WORDS

    def compact(t, off):
        a = addr_v[pl.ds(t * 16, 16)] - base0
        v = val_v[pl.ds(t * 16, 16)]
        m = (a >= 0) & (a < W_WORDS)
        plsc.store_compressed(ca_v.at[pl.ds(off, 16)], a, mask=m)
        plsc.store_compressed(cv_v.at[pl.ds(off, 16)], v, mask=m)
        return off + jnp.sum(m.astype(jnp.int32))

    m_w = lax.fori_loop(0, NENT // 16, compact, 0)
    n_c = (m_w + 15) >> 4

    for p in range(NPASS):
        base = p * PASS_WORDS
        pltpu.sync_copy(zeros_hbm, buf)

        def chunk(t, carry, base=base):
            a = ca_v[pl.ds(t * 16, 16)] - base
            v = cv_v[pl.ds(t * 16, 16)]
            lane = lax.iota(jnp.int32, 16) + t * 16
            m = (a >= 0) & (a < PASS_WORDS) & (lane < m_w)
            plsc.addupdate_scatter(buf, [jnp.where(m, a, 0)], v, mask=m)
            return carry

        lax.fori_loop(0, n_c, chunk, 0)
        pltpu.sync_copy(buf, out_hbm.at[pl.ds(base0 + base, PASS_WORDS)])


@functools.lru_cache(maxsize=1)
def _scatter_sym():
    return pl.kernel(
        _scatter_body,
        out_type=jax.ShapeDtypeStruct((WORDS,), jnp.float32),
        mesh=plsc.VectorSubcoreMesh(core_axis_name="c", subcore_axis_name="s"),
        scratch_types=[
            pltpu.VMEM((NENT,), jnp.int32),
            pltpu.VMEM((NENT,), jnp.float32),
            pltpu.VMEM((NENT + 32,), jnp.int32),
            pltpu.VMEM((NENT + 32,), jnp.float32),
            pltpu.VMEM((PASS_WORDS,), jnp.float32),
        ],
        compiler_params=pltpu.CompilerParams(needs_layout_passes=False),
    )


def _pairs_body(skey_h, sidx_h, r_h, c_h, pk_h, cnt_h,
                skey_v, sidx_v, r_v, c_v, cb_v, cnt_v):
    cid = lax.axis_index("c")
    sid = lax.axis_index("s")
    wid = sid * 2 + cid
    pltpu.sync_copy(skey_h, skey_v)
    pltpu.sync_copy(sidx_h, sidx_v)
    pltpu.sync_copy(r_h, r_v)
    pltpu.sync_copy(c_h, c_v)
    iota = lax.iota(jnp.int32, 16)
    base_c = wid * CAND_W

    def chunk(t, off):
        cand = base_c + t * 16 + iota
        k = cand >> 13
        i = cand & (NNZ - 1)
        kc = jnp.where(k == 9, 4, k)
        d3 = (kc * 11) >> 5
        dy = d3 - 1
        dx = kc - d3 * 3 - 1
        rr = plsc.load_gather(r_v, [i]) + dy
        cc = plsc.load_gather(c_v, [i]) + dx
        inb = (rr >= 0) & (rr < NG) & (cc >= 0) & (cc < NG)
        q = rr * NG + cc
        lo = jnp.zeros((16,), jnp.int32)
        hi = jnp.full((16,), NNZ, jnp.int32)
        for _ in range(13):
            mid = (lo + hi) >> 1
            v = plsc.load_gather(skey_v, [mid])
            le = v <= q
            lo = jnp.where(le, mid + 1, lo)
            hi = jnp.where(le, hi, mid)
        pos = lo - 1
        posc = jnp.maximum(pos, 0)
        match = (pos >= 0) & (plsc.load_gather(skey_v, [posc]) == q)
        nb = plsc.load_gather(sidx_v, [posc])
        notc = (k != 4) & (k != 9)
        valid = inb & match & (notc | (nb != i))
        jf = jnp.where(k == 9, i, nb)
        kk = jnp.where(k == 9, 13, k)
        code = i | (jnp.where(valid, jf, 0) << 13) | (kk << 26)
        plsc.store_compressed(cb_v.at[pl.ds(off, 16)], code, mask=valid)
        return off + jnp.sum(valid.astype(jnp.int32))

    off = lax.fori_loop(0, CAND_W // 16, chunk, 0)
    pltpu.sync_copy(cb_v, pk_h.at[wid])
    cnt_v[...] = jnp.zeros((16,), jnp.int32) + off
    pltpu.sync_copy(cnt_v, cnt_h.at[wid])


@functools.lru_cache(maxsize=1)
def _pairs_sc():
    return pl.kernel(
        _pairs_body,
        out_type=(jax.ShapeDtypeStruct((NWK, SEG), jnp.int32),
                  jax.ShapeDtypeStruct((NWK, 16), jnp.int32)),
        mesh=plsc.VectorSubcoreMesh(core_axis_name="c", subcore_axis_name="s"),
        scratch_types=[
            pltpu.VMEM((NNZ,), jnp.int32),
            pltpu.VMEM((NNZ,), jnp.int32),
            pltpu.VMEM((NNZ,), jnp.int32),
            pltpu.VMEM((NNZ,), jnp.int32),
            pltpu.VMEM((SEG,), jnp.int32),
            pltpu.VMEM((16,), jnp.int32),
        ],
        compiler_params=pltpu.CompilerParams(needs_layout_passes=False),
    )


def _structure(r, c):
    ar = jnp.arange(NNZ, dtype=jnp.int32)
    key = r * NG + c
    skey, sidx = lax.sort_key_val(key, ar, is_stable=True)
    pk, cnts = _pairs_sc()(skey, sidx, r, c)
    return pk, cnts[:, 0]


def _network(A_values, pk, cnt, W1, b1, a1, W2, b2, a2, W3, b3, a3,
             W4, b4, a4, W5, b5, a5, W6, b6, a6):
    av = A_values.reshape(NNZ, 1)
    x2 = _layer12(av, W1.reshape(1, 64), b1.reshape(1, 64),
                  a1.reshape(1), W2, b2.reshape(1, 256), a2.reshape(1),
                  cnt, pk)
    x3 = _conv_layer(x2, W3, b3.reshape(1, 512), a3.reshape(1), cnt, pk,
                     256, 512)
    x4 = _conv_layer(x3, W4, b4.reshape(1, 256), a4.reshape(1), cnt, pk,
                     512, 256)
    x5 = _conv_layer(x4, W5, b5.reshape(1, 64), a5.reshape(1), cnt, pk,
                     256, 64)
    x6 = _conv_layer(x5, W6, b6.reshape(1, 1), a6.reshape(1), cnt, pk,
                     64, 1)
    return x6.reshape(NNZ)


def kernel(A_values, A_indices, W1, b1, a1, W2, b2, a2, W3, b3, a3,
           W4, b4, a4, W5, b5, a5, W6, b6, a6):
    r = A_indices[0].astype(jnp.int32)
    c = A_indices[1].astype(jnp.int32)
    pk, cnt = _structure(r, c)
    x6 = _network(A_values, pk, cnt, W1, b1, a1, W2, b2, a2, W3, b3, a3,
                  W4, b4, a4, W5, b5, a5, W6, b6, a6)
    addr = jnp.concatenate([r * NG + c, c * NG + r]).astype(jnp.int32)
    vals = 0.5 * jnp.concatenate([x6, x6])
    zeros = jnp.zeros((PASS_WORDS,), jnp.float32)
    flat = _scatter_sym()(addr, vals, zeros)
    return flat.reshape(NG, NG)


# speedup vs baseline: 1.1394x; 1.1394x over previous
import functools

import jax
import jax.numpy as jnp
from jax import lax
from jax.experimental import pallas as pl
from jax.experimental.pallas import tpu as pltpu
from jax.experimental.pallas import tpu_sc as plsc

NG = 4096
NNZ = 8192
NWK = 32
NCAND = 10 * NNZ
CAND_W = NCAND // NWK
SEG = CAND_W + 32


P_CH = 128


def _apply_pairs(x_ref, w_ref, cnt_ref, pk_ref, y_ref, xg_ref, cm_ref, ci, co):
    def seg(w, carry):
        cw = cnt_ref[w]
        n_ch = (cw + P_CH - 1) // P_CH

        def chunk(ch, carry2):
            m = jnp.minimum(cw - ch * P_CH, P_CH)

            def gather(p, carry3):
                code = pk_ref[w, ch * P_CH + p]
                j = (code >> 13) & (NNZ - 1)
                xg_ref[pl.ds(p, 1), :] = x_ref[pl.ds(j, 1), :]
                return carry3

            lax.fori_loop(0, m, gather, 0)
            for k in range(9):
                cm_ref[pl.ds(k * P_CH, P_CH), :] = jnp.dot(
                    xg_ref[...], w_ref[k], preferred_element_type=jnp.float32)

            def scatter(p, carry3):
                code = pk_ref[w, ch * P_CH + p]
                i = code & (NNZ - 1)
                kk = code >> 26
                neg = kk >= 9
                k = jnp.where(neg, kk - 9, kk)
                s = jnp.where(neg, jnp.float32(-1.0), jnp.float32(1.0))
                y_ref[pl.ds(i, 1), :] = (y_ref[pl.ds(i, 1), :]
                                         + s * cm_ref[pl.ds(k * P_CH + p, 1), :])
                return carry3

            lax.fori_loop(0, m, scatter, 0)
            return carry2

        lax.fori_loop(0, n_ch, chunk, 0)
        return carry

    lax.fori_loop(0, NWK, seg, 0)


def _prelu_ref(y_ref, a_ref):
    a = a_ref[0]
    yv = y_ref[...]
    y_ref[...] = jnp.where(yv >= 0, yv, a * yv)


def _net_body(av_ref, w1_ref, b1_ref, a1_ref, w2_ref, b2_ref, a2_ref,
              w3_ref, b3_ref, a3_ref, w4_ref, b4_ref, a4_ref,
              w5_ref, b5_ref, a5_ref, w6_ref, b6_ref, a6_ref,
              cnt_ref, pk_ref, out_ref,
              xa_s, xb_s, xc_s, xg64_s, xg256_s, xg512_s,
              cm512_s, cm256_s, cm64_s, cm1_s):
    x1 = av_ref[...] * w1_ref[...] + b1_ref[...]
    a1 = a1_ref[0]
    xa_s[...] = jnp.where(x1 >= 0, x1, a1 * x1)

    def conv(x_ref, w_ref, b_ref, a_ref, y_ref, xg_ref, cm_ref):
        y_ref[...] = (jnp.dot(x_ref[...], w_ref[4],
                              preferred_element_type=jnp.float32) + b_ref[...])
        _apply_pairs(x_ref, w_ref, cnt_ref, pk_ref, y_ref, xg_ref, cm_ref, 0, 0)
        _prelu_ref(y_ref, a_ref)

    conv(xa_s, w2_ref, b2_ref, a2_ref, xb_s, xg64_s, cm256_s)
    conv(xb_s, w3_ref, b3_ref, a3_ref, xc_s, xg256_s, cm512_s)
    conv(xc_s, w4_ref, b4_ref, a4_ref, xb_s, xg512_s, cm256_s)
    conv(xb_s, w5_ref, b5_ref, a5_ref, xa_s, xg256_s, cm64_s)
    conv(xa_s, w6_ref, b6_ref, a6_ref, out_ref, xg64_s, cm1_s)


_SMEM = pl.BlockSpec(memory_space=pltpu.SMEM)


def _vspec():
    return pl.BlockSpec(memory_space=pltpu.VMEM)


def _run_net(av, cnt, pk, W1, b1, a1, W2, b2, a2, W3, b3, a3,
             W4, b4, a4, W5, b5, a5, W6, b6, a6):
    return pl.pallas_call(
        _net_body,
        out_shape=jax.ShapeDtypeStruct((NNZ, 1), jnp.float32),
        in_specs=[_vspec(), _vspec(), _vspec(), _SMEM,
                  _vspec(), _vspec(), _SMEM,
                  _vspec(), _vspec(), _SMEM,
                  _vspec(), _vspec(), _SMEM,
                  _vspec(), _vspec(), _SMEM,
                  _vspec(), _vspec(), _SMEM,
                  _SMEM, _SMEM],
        out_specs=_vspec(),
        scratch_shapes=[pltpu.VMEM((NNZ, 64), jnp.float32),
                        pltpu.VMEM((NNZ, 256), jnp.float32),
                        pltpu.VMEM((NNZ, 512), jnp.float32),
                        pltpu.VMEM((P_CH, 64), jnp.float32),
                        pltpu.VMEM((P_CH, 256), jnp.float32),
                        pltpu.VMEM((P_CH, 512), jnp.float32),
                        pltpu.VMEM((9 * P_CH, 512), jnp.float32),
                        pltpu.VMEM((9 * P_CH, 256), jnp.float32),
                        pltpu.VMEM((9 * P_CH, 64), jnp.float32),
                        pltpu.VMEM((9 * P_CH, 1), jnp.float32)],
    )(av, W1.reshape(1, 64), b1.reshape(1, 64), a1.reshape(1),
      W2, b2.reshape(1, 256), a2.reshape(1),
      W3, b3.reshape(1, 512), a3.reshape(1),
      W4, b4.reshape(1, 256), a4.reshape(1),
      W5, b5.reshape(1, 64), a5.reshape(1),
      W6, b6.reshape(1, 1), a6.reshape(1),
      cnt, pk)


WORDS = NG * NG
W_WORDS = WORDS // NWK
PASS_WORDS = 32768
NPASS = W_WORDS // PASS_WORDS
NENT = 2 * NNZ


def _scatter_body(addr_hbm, val_hbm, zeros_hbm, out_hbm,
                  addr_v, val_v, ca_v, cv_v, buf):
    cid = lax.axis_index("c")
    sid = lax.axis_index("s")
    wid = sid * 2 + cid
    pltpu.sync_copy(addr_hbm, addr_v)
    pltpu.sync_copy(val_hbm, val_v)
    base0 = wid * W_WORDS

    def compact(t, off):
        a = addr_v[pl.ds(t * 16, 16)] - base0
        v = val_v[pl.ds(t * 16, 16)]
        m = (a >= 0) & (a < W_WORDS)
        plsc.store_compressed(ca_v.at[pl.ds(off, 16)], a, mask=m)
        plsc.store_compressed(cv_v.at[pl.ds(off, 16)], v, mask=m)
        return off + jnp.sum(m.astype(jnp.int32))

    m_w = lax.fori_loop(0, NENT // 16, compact, 0)
    n_c = (m_w + 15) >> 4

    for p in range(NPASS):
        base = p * PASS_WORDS
        pltpu.sync_copy(zeros_hbm, buf)

        def chunk(t, carry, base=base):
            a = ca_v[pl.ds(t * 16, 16)] - base
            v = cv_v[pl.ds(t * 16, 16)]
            lane = lax.iota(jnp.int32, 16) + t * 16
            m = (a >= 0) & (a < PASS_WORDS) & (lane < m_w)
            plsc.addupdate_scatter(buf, [jnp.where(m, a, 0)], v, mask=m)
            return carry

        lax.fori_loop(0, n_c, chunk, 0)
        pltpu.sync_copy(buf, out_hbm.at[pl.ds(base0 + base, PASS_WORDS)])


@functools.lru_cache(maxsize=1)
def _scatter_sym():
    return pl.kernel(
        _scatter_body,
        out_type=jax.ShapeDtypeStruct((WORDS,), jnp.float32),
        mesh=plsc.VectorSubcoreMesh(core_axis_name="c", subcore_axis_name="s"),
        scratch_types=[
            pltpu.VMEM((NENT,), jnp.int32),
            pltpu.VMEM((NENT,), jnp.float32),
            pltpu.VMEM((NENT + 32,), jnp.int32),
            pltpu.VMEM((NENT + 32,), jnp.float32),
            pltpu.VMEM((PASS_WORDS,), jnp.float32),
        ],
        compiler_params=pltpu.CompilerParams(needs_layout_passes=False),
    )


def _pairs_body(skey_h, sidx_h, r_h, c_h, pk_h, cnt_h,
                skey_v, sidx_v, r_v, c_v, cb_v, cnt_v):
    cid = lax.axis_index("c")
    sid = lax.axis_index("s")
    wid = sid * 2 + cid
    pltpu.sync_copy(skey_h, skey_v)
    pltpu.sync_copy(sidx_h, sidx_v)
    pltpu.sync_copy(r_h, r_v)
    pltpu.sync_copy(c_h, c_v)
    iota = lax.iota(jnp.int32, 16)
    base_c = wid * CAND_W

    def chunk(t, off):
        cand = base_c + t * 16 + iota
        k = cand >> 13
        i = cand & (NNZ - 1)
        kc = jnp.where(k == 9, 4, k)
        d3 = (kc * 11) >> 5
        dy = d3 - 1
        dx = kc - d3 * 3 - 1
        rr = plsc.load_gather(r_v, [i]) + dy
        cc = plsc.load_gather(c_v, [i]) + dx
        inb = (rr >= 0) & (rr < NG) & (cc >= 0) & (cc < NG)
        q = rr * NG + cc
        lo = jnp.zeros((16,), jnp.int32)
        hi = jnp.full((16,), NNZ, jnp.int32)
        for _ in range(13):
            mid = (lo + hi) >> 1
            v = plsc.load_gather(skey_v, [mid])
            le = v <= q
            lo = jnp.where(le, mid + 1, lo)
            hi = jnp.where(le, hi, mid)
        pos = lo - 1
        posc = jnp.maximum(pos, 0)
        match = (pos >= 0) & (plsc.load_gather(skey_v, [posc]) == q)
        nb = plsc.load_gather(sidx_v, [posc])
        notc = (k != 4) & (k != 9)
        valid = inb & match & (notc | (nb != i))
        jf = jnp.where(k == 9, i, nb)
        kk = jnp.where(k == 9, 13, k)
        code = i | (jnp.where(valid, jf, 0) << 13) | (kk << 26)
        plsc.store_compressed(cb_v.at[pl.ds(off, 16)], code, mask=valid)
        return off + jnp.sum(valid.astype(jnp.int32))

    off = lax.fori_loop(0, CAND_W // 16, chunk, 0)
    pltpu.sync_copy(cb_v, pk_h.at[wid])
    cnt_v[...] = jnp.zeros((16,), jnp.int32) + off
    pltpu.sync_copy(cnt_v, cnt_h.at[wid])


@functools.lru_cache(maxsize=1)
def _pairs_sc():
    return pl.kernel(
        _pairs_body,
        out_type=(jax.ShapeDtypeStruct((NWK, SEG), jnp.int32),
                  jax.ShapeDtypeStruct((NWK, 16), jnp.int32)),
        mesh=plsc.VectorSubcoreMesh(core_axis_name="c", subcore_axis_name="s"),
        scratch_types=[
            pltpu.VMEM((NNZ,), jnp.int32),
            pltpu.VMEM((NNZ,), jnp.int32),
            pltpu.VMEM((NNZ,), jnp.int32),
            pltpu.VMEM((NNZ,), jnp.int32),
            pltpu.VMEM((SEG,), jnp.int32),
            pltpu.VMEM((16,), jnp.int32),
        ],
        compiler_params=pltpu.CompilerParams(needs_layout_passes=False),
    )


def _structure(r, c):
    ar = jnp.arange(NNZ, dtype=jnp.int32)
    key = r * NG + c
    skey, sidx = lax.sort_key_val(key, ar, is_stable=True)
    pk, cnts = _pairs_sc()(skey, sidx, r, c)
    return pk, cnts[:, 0]


def _network(A_values, pk, cnt, W1, b1, a1, W2, b2, a2, W3, b3, a3,
             W4, b4, a4, W5, b5, a5, W6, b6, a6):
    av = A_values.reshape(NNZ, 1)
    x6 = _run_net(av, cnt, pk, W1, b1, a1, W2, b2, a2, W3, b3, a3,
                  W4, b4, a4, W5, b5, a5, W6, b6, a6)
    return x6.reshape(NNZ)


def kernel(A_values, A_indices, W1, b1, a1, W2, b2, a2, W3, b3, a3,
           W4, b4, a4, W5, b5, a5, W6, b6, a6):
    r = A_indices[0].astype(jnp.int32)
    c = A_indices[1].astype(jnp.int32)
    pk, cnt = _structure(r, c)
    x6 = _network(A_values, pk, cnt, W1, b1, a1, W2, b2, a2, W3, b3, a3,
                  W4, b4, a4, W5, b5, a5, W6, b6, a6)
    addr = jnp.concatenate([r * NG + c, c * NG + r]).astype(jnp.int32)
    vals = 0.5 * jnp.concatenate([x6, x6])
    zeros = jnp.zeros((PASS_WORDS,), jnp.float32)
    flat = _scatter_sym()(addr, vals, zeros)
    return flat.reshape(NG, NG)


# speedup vs baseline: 1.5215x; 1.3354x over previous
import functools

import jax
import jax.numpy as jnp
from jax import lax
from jax.experimental import pallas as pl
from jax.experimental.pallas import tpu as pltpu
from jax.experimental.pallas import tpu_sc as plsc

NG = 4096
NNZ = 8192
NWK = 32
NCAND = 10 * NNZ
CAND_W = NCAND // NWK
SEG = CAND_W + 32


P_CH = 128


def _apply_pairs(x_ref, w_ref, cnt_ref, pk_ref, y_ref, xg_ref, cm_ref, ci, co):
    def seg(w, carry):
        cw = cnt_ref[w]
        n_ch = (cw + P_CH - 1) // P_CH

        def chunk(ch, carry2):
            m = jnp.minimum(cw - ch * P_CH, P_CH)

            def gather(p, carry3):
                code = pk_ref[w, ch * P_CH + p]
                j = (code >> 13) & (NNZ - 1)
                xg_ref[pl.ds(p, 1), :] = x_ref[pl.ds(j, 1), :]
                return carry3

            lax.fori_loop(0, m, gather, 0)
            for k in range(9):
                cm_ref[pl.ds(k * P_CH, P_CH), :] = jnp.dot(
                    xg_ref[...], w_ref[k], preferred_element_type=jnp.float32)

            def scatter(p, carry3):
                code = pk_ref[w, ch * P_CH + p]
                i = code & (NNZ - 1)
                kk = code >> 26
                neg = kk >= 9
                k = jnp.where(neg, kk - 9, kk)
                s = jnp.where(neg, jnp.float32(-1.0), jnp.float32(1.0))
                y_ref[pl.ds(i, 1), :] = (y_ref[pl.ds(i, 1), :]
                                         + s * cm_ref[pl.ds(k * P_CH + p, 1), :])
                return carry3

            lax.fori_loop(0, m, scatter, 0)
            return carry2

        lax.fori_loop(0, n_ch, chunk, 0)
        return carry

    lax.fori_loop(0, NWK, seg, 0)


def _prelu_ref(y_ref, a_ref):
    a = a_ref[0]
    yv = y_ref[...]
    y_ref[...] = jnp.where(yv >= 0, yv, a * yv)


def _net_body(av_ref, w1_ref, b1_ref, a1_ref, w2_ref, b2_ref, a2_ref,
              w3_ref, b3_ref, a3_ref, w4_ref, b4_ref, a4_ref,
              w5_ref, b5_ref, a5_ref, w6_ref, b6_ref, a6_ref,
              cnt_ref, pk_ref, out_ref,
              xa_s, xb_s, xc_s, xg64_s, xg256_s, xg512_s,
              cm512_s, cm256_s, cm64_s, cm1_s):
    x1 = av_ref[...] * w1_ref[...] + b1_ref[...]
    a1 = a1_ref[0]
    xa_s[...] = jnp.where(x1 >= 0, x1, a1 * x1)

    def conv(x_ref, w_ref, b_ref, a_ref, y_ref, xg_ref, cm_ref):
        y_ref[...] = (jnp.dot(x_ref[...], w_ref[4],
                              preferred_element_type=jnp.float32) + b_ref[...])
        _apply_pairs(x_ref, w_ref, cnt_ref, pk_ref, y_ref, xg_ref, cm_ref, 0, 0)
        _prelu_ref(y_ref, a_ref)

    conv(xa_s, w2_ref, b2_ref, a2_ref, xb_s, xg64_s, cm256_s)
    conv(xb_s, w3_ref, b3_ref, a3_ref, xc_s, xg256_s, cm512_s)
    conv(xc_s, w4_ref, b4_ref, a4_ref, xb_s, xg512_s, cm256_s)
    conv(xb_s, w5_ref, b5_ref, a5_ref, xa_s, xg256_s, cm64_s)
    conv(xa_s, w6_ref, b6_ref, a6_ref, out_ref, xg64_s, cm1_s)


_SMEM = pl.BlockSpec(memory_space=pltpu.SMEM)


def _vspec():
    return pl.BlockSpec(memory_space=pltpu.VMEM)


def _run_net(av, cnt, pk, W1, b1, a1, W2, b2, a2, W3, b3, a3,
             W4, b4, a4, W5, b5, a5, W6, b6, a6):
    return pl.pallas_call(
        _net_body,
        out_shape=jax.ShapeDtypeStruct((NNZ, 1), jnp.float32),
        in_specs=[_vspec(), _vspec(), _vspec(), _SMEM,
                  _vspec(), _vspec(), _SMEM,
                  _vspec(), _vspec(), _SMEM,
                  _vspec(), _vspec(), _SMEM,
                  _vspec(), _vspec(), _SMEM,
                  _vspec(), _vspec(), _SMEM,
                  _SMEM, _SMEM],
        out_specs=_vspec(),
        scratch_shapes=[pltpu.VMEM((NNZ, 64), jnp.float32),
                        pltpu.VMEM((NNZ, 256), jnp.float32),
                        pltpu.VMEM((NNZ, 512), jnp.float32),
                        pltpu.VMEM((P_CH, 64), jnp.float32),
                        pltpu.VMEM((P_CH, 256), jnp.float32),
                        pltpu.VMEM((P_CH, 512), jnp.float32),
                        pltpu.VMEM((9 * P_CH, 512), jnp.float32),
                        pltpu.VMEM((9 * P_CH, 256), jnp.float32),
                        pltpu.VMEM((9 * P_CH, 64), jnp.float32),
                        pltpu.VMEM((9 * P_CH, 1), jnp.float32)],
    )(av, W1.reshape(1, 64), b1.reshape(1, 64), a1.reshape(1),
      W2, b2.reshape(1, 256), a2.reshape(1),
      W3, b3.reshape(1, 512), a3.reshape(1),
      W4, b4.reshape(1, 256), a4.reshape(1),
      W5, b5.reshape(1, 64), a5.reshape(1),
      W6, b6.reshape(1, 1), a6.reshape(1),
      cnt, pk)


WORDS = NG * NG
W_WORDS = WORDS // NWK
PASS_WORDS = 32768
NPASS = W_WORDS // PASS_WORDS
NENT = 2 * NNZ


def _scatter_body(addr_hbm, val_hbm, out_hbm,
                  addr_v, val_v, ca_v, cv_v, buf):
    cid = lax.axis_index("c")
    sid = lax.axis_index("s")
    wid = sid * 2 + cid
    pltpu.sync_copy(addr_hbm, addr_v)
    pltpu.sync_copy(val_hbm, val_v)
    base0 = wid * W_WORDS

    def compact(t, off):
        a = addr_v[pl.ds(t * 16, 16)] - base0
        v = val_v[pl.ds(t * 16, 16)]
        m = (a >= 0) & (a < W_WORDS)
        plsc.store_compressed(ca_v.at[pl.ds(off, 16)], a, mask=m)
        plsc.store_compressed(cv_v.at[pl.ds(off, 16)], v, mask=m)
        return off + jnp.sum(m.astype(jnp.int32))

    m_w = lax.fori_loop(0, NENT // 16, compact, 0)
    n_c = (m_w + 15) >> 4
    z16 = jnp.zeros((16,), jnp.float32)

    def zfill(t, carry):
        buf[pl.ds(t * 16, 16)] = z16
        return carry

    lax.fori_loop(0, PASS_WORDS // 16, zfill, 0)

    for p in range(NPASS):
        base = p * PASS_WORDS

        def chunk(t, carry, base=base):
            a = ca_v[pl.ds(t * 16, 16)] - base
            v = cv_v[pl.ds(t * 16, 16)]
            lane = lax.iota(jnp.int32, 16) + t * 16
            m = (a >= 0) & (a < PASS_WORDS) & (lane < m_w)
            plsc.addupdate_scatter(buf, [jnp.where(m, a, 0)], v, mask=m)
            return carry

        lax.fori_loop(0, n_c, chunk, 0)
        pltpu.sync_copy(buf, out_hbm.at[pl.ds(base0 + base, PASS_WORDS)])

        def unchunk(t, carry, base=base):
            a = ca_v[pl.ds(t * 16, 16)] - base
            lane = lax.iota(jnp.int32, 16) + t * 16
            m = (a >= 0) & (a < PASS_WORDS) & (lane < m_w)
            plsc.store_scatter(buf, [jnp.where(m, a, 0)], z16, mask=m)
            return carry

        lax.fori_loop(0, n_c, unchunk, 0)


@functools.lru_cache(maxsize=1)
def _scatter_sym():
    return pl.kernel(
        _scatter_body,
        out_type=jax.ShapeDtypeStruct((WORDS,), jnp.float32),
        mesh=plsc.VectorSubcoreMesh(core_axis_name="c", subcore_axis_name="s"),
        scratch_types=[
            pltpu.VMEM((NENT,), jnp.int32),
            pltpu.VMEM((NENT,), jnp.float32),
            pltpu.VMEM((NENT + 32,), jnp.int32),
            pltpu.VMEM((NENT + 32,), jnp.float32),
            pltpu.VMEM((PASS_WORDS,), jnp.float32),
        ],
        compiler_params=pltpu.CompilerParams(needs_layout_passes=False),
    )


def _pairs_body(skey_h, sidx_h, r_h, c_h, pk_h, cnt_h,
                skey_v, sidx_v, r_v, c_v, cb_v, cnt_v):
    cid = lax.axis_index("c")
    sid = lax.axis_index("s")
    wid = sid * 2 + cid
    pltpu.sync_copy(skey_h, skey_v)
    pltpu.sync_copy(sidx_h, sidx_v)
    pltpu.sync_copy(r_h, r_v)
    pltpu.sync_copy(c_h, c_v)
    iota = lax.iota(jnp.int32, 16)
    base_c = wid * CAND_W

    def chunk(t, off):
        cand = base_c + t * 16 + iota
        k = cand >> 13
        i = cand & (NNZ - 1)
        kc = jnp.where(k == 9, 4, k)
        d3 = (kc * 11) >> 5
        dy = d3 - 1
        dx = kc - d3 * 3 - 1
        rr = plsc.load_gather(r_v, [i]) + dy
        cc = plsc.load_gather(c_v, [i]) + dx
        inb = (rr >= 0) & (rr < NG) & (cc >= 0) & (cc < NG)
        q = rr * NG + cc
        lo = jnp.zeros((16,), jnp.int32)
        hi = jnp.full((16,), NNZ, jnp.int32)
        for _ in range(13):
            mid = (lo + hi) >> 1
            v = plsc.load_gather(skey_v, [mid])
            le = v <= q
            lo = jnp.where(le, mid + 1, lo)
            hi = jnp.where(le, hi, mid)
        pos = lo - 1
        posc = jnp.maximum(pos, 0)
        match = (pos >= 0) & (plsc.load_gather(skey_v, [posc]) == q)
        nb = plsc.load_gather(sidx_v, [posc])
        notc = (k != 4) & (k != 9)
        valid = inb & match & (notc | (nb != i))
        jf = jnp.where(k == 9, i, nb)
        kk = jnp.where(k == 9, 13, k)
        code = i | (jnp.where(valid, jf, 0) << 13) | (kk << 26)
        plsc.store_compressed(cb_v.at[pl.ds(off, 16)], code, mask=valid)
        return off + jnp.sum(valid.astype(jnp.int32))

    off = lax.fori_loop(0, CAND_W // 16, chunk, 0)
    pltpu.sync_copy(cb_v, pk_h.at[wid])
    cnt_v[...] = jnp.zeros((16,), jnp.int32) + off
    pltpu.sync_copy(cnt_v, cnt_h.at[wid])


@functools.lru_cache(maxsize=1)
def _pairs_sc():
    return pl.kernel(
        _pairs_body,
        out_type=(jax.ShapeDtypeStruct((NWK, SEG), jnp.int32),
                  jax.ShapeDtypeStruct((NWK, 16), jnp.int32)),
        mesh=plsc.VectorSubcoreMesh(core_axis_name="c", subcore_axis_name="s"),
        scratch_types=[
            pltpu.VMEM((NNZ,), jnp.int32),
            pltpu.VMEM((NNZ,), jnp.int32),
            pltpu.VMEM((NNZ,), jnp.int32),
            pltpu.VMEM((NNZ,), jnp.int32),
            pltpu.VMEM((SEG,), jnp.int32),
            pltpu.VMEM((16,), jnp.int32),
        ],
        compiler_params=pltpu.CompilerParams(needs_layout_passes=False),
    )


def _structure(r, c):
    ar = jnp.arange(NNZ, dtype=jnp.int32)
    key = r * NG + c
    skey, sidx = lax.sort_key_val(key, ar, is_stable=True)
    pk, cnts = _pairs_sc()(skey, sidx, r, c)
    return pk, cnts[:, 0]


def _network(A_values, pk, cnt, W1, b1, a1, W2, b2, a2, W3, b3, a3,
             W4, b4, a4, W5, b5, a5, W6, b6, a6):
    av = A_values.reshape(NNZ, 1)
    x6 = _run_net(av, cnt, pk, W1, b1, a1, W2, b2, a2, W3, b3, a3,
                  W4, b4, a4, W5, b5, a5, W6, b6, a6)
    return x6.reshape(NNZ)


def kernel(A_values, A_indices, W1, b1, a1, W2, b2, a2, W3, b3, a3,
           W4, b4, a4, W5, b5, a5, W6, b6, a6):
    r = A_indices[0].astype(jnp.int32)
    c = A_indices[1].astype(jnp.int32)
    pk, cnt = _structure(r, c)
    x6 = _network(A_values, pk, cnt, W1, b1, a1, W2, b2, a2, W3, b3, a3,
                  W4, b4, a4, W5, b5, a5, W6, b6, a6)
    addr = jnp.concatenate([r * NG + c, c * NG + r]).astype(jnp.int32)
    vals = 0.5 * jnp.concatenate([x6, x6])
    flat = _scatter_sym()(addr, vals)
    return flat.reshape(NG, NG)
